# baseline (speedup 1.0000x reference)
import functools

import jax
import jax.numpy as jnp
from jax import lax
from jax.experimental import pallas as pl
from jax.experimental.pallas import tpu as pltpu
from jax.experimental.pallas import tpu_sc as plsc

NUM_EMBEDDINGS = 1000000
EMBED_DIM = 64
BATCH = 16384
HIST = 50

NC = 2
NS = 16
NW = NC * NS

B = BATCH * HIST
NBLK = BATCH // 128
NUNIT = HIST * NBLK
U_PER_W = NUNIT // NW
CHUNK = 128


def _make_kernel():
    mesh = plsc.VectorSubcoreMesh(core_axis_name="c", subcore_axis_name="s")

    @functools.partial(
        pl.kernel,
        mesh=mesh,
        out_type=jax.ShapeDtypeStruct((HIST, 8, NBLK, 1024), jnp.float32),
        scratch_types=[
            pltpu.VMEM((U_PER_W, CHUNK), jnp.int32),
            [pltpu.VMEM((CHUNK, 128), jnp.float32) for _ in range(2)],
            [pltpu.VMEM((8, 1024), jnp.float32) for _ in range(2)],
            [pltpu.SemaphoreType.DMA for _ in range(2)],
            [pltpu.SemaphoreType.DMA for _ in range(2)],
        ],
        compiler_params=pltpu.CompilerParams(
            use_tc_tiling_on_sc=False, needs_layout_passes=False
        ),
    )
    def emb_gather(idx_hbm, table_hbm, out_hbm, idx_v, rows, trs, sem_g, sem_w):
        wid = lax.axis_index("s") * NC + lax.axis_index("c")
        u0 = wid * U_PER_W
        pltpu.sync_copy(idx_hbm.at[wid], idx_v)

        trow = [kk * 16 + lax.iota(jnp.int32, 16) for kk in range(8)]

        def fire_gather(j, b):
            pltpu.async_copy(table_hbm.at[idx_v.at[j]], rows[b], sem_g[b])

        def wait_gather(b):
            pltpu.make_async_copy(
                table_hbm.at[idx_v.at[0]], rows[b], sem_g[b]
            ).wait()

        def fire_writeback(u, b):
            h = u // NBLK
            c = u % NBLK
            pltpu.async_copy(trs[b], out_hbm.at[h, :, c], sem_w[b])

        def wait_writeback(b):
            pltpu.make_async_copy(trs[b], out_hbm.at[0, :, 0], sem_w[b]).wait()

        def transpose_unit(b):
            def d_body(d, carry):
                tcol = lax.broadcast(d, (16,))
                r = d // 8
                s = d % 8
                for kk in range(8):
                    vals = plsc.load_gather(rows[b], [trow[kk], tcol])
                    trs[b][r, pl.ds(s * 128 + kk * 16, 16)] = vals
                return carry

            lax.fori_loop(0, 64, d_body, 0, unroll=2)

        fire_gather(0, 0)
        fire_gather(1, 1)

        def pair_body(g, carry):
            for b in range(2):
                j = g * 2 + b
                wait_gather(b)

                @pl.when(j >= 2)
                def _():
                    wait_writeback(b)

                transpose_unit(b)
                fire_writeback(u0 + j, b)

                @pl.when(j + 2 < U_PER_W)
                def _():
                    fire_gather(j + 2, b)
            return carry

        lax.fori_loop(0, U_PER_W // 2, pair_body, 0)
        wait_writeback(0)
        wait_writeback(1)

    return emb_gather


_emb_gather = _make_kernel()


@jax.jit
def kernel(token_ids, lookup_table):
    tbl_pad = jnp.pad(lookup_table, ((0, 0), (0, 64)))
    tok3 = token_ids.T.reshape(NW, U_PER_W, CHUNK)
    out5 = _emb_gather(tok3, tbl_pad)
    out = (
        out5.reshape(HIST, 8, NBLK, 8, 128)
        .transpose(2, 4, 0, 1, 3)
        .reshape(BATCH, HIST, EMBED_DIM)
    )
    return out


# speedup vs baseline: 1.4840x; 1.4840x over previous
import functools

import jax
import jax.numpy as jnp
from jax import lax
from jax.experimental import pallas as pl
from jax.experimental.pallas import tpu as pltpu
from jax.experimental.pallas import tpu_sc as plsc

NUM_EMBEDDINGS = 1000000
EMBED_DIM = 64
BATCH = 16384
HIST = 50

NC = 2
NS = 16
NW = NC * NS

B = BATCH * HIST
NBLK = BATCH // 128
NUNIT = HIST * NBLK
U_PER_W = NUNIT // NW
CHUNK = 128


def _make_kernel():
    mesh = plsc.VectorSubcoreMesh(core_axis_name="c", subcore_axis_name="s")

    @functools.partial(
        pl.kernel,
        mesh=mesh,
        out_type=jax.ShapeDtypeStruct((HIST, 8, NBLK, 1024), jnp.float32),
        scratch_types=[
            pltpu.VMEM((U_PER_W, CHUNK), jnp.int32),
            [pltpu.VMEM((CHUNK, 128), jnp.float32) for _ in range(2)],
            [pltpu.VMEM((8, 1024), jnp.float32) for _ in range(2)],
            [pltpu.SemaphoreType.DMA for _ in range(2)],
            [pltpu.SemaphoreType.DMA for _ in range(2)],
        ],
        compiler_params=pltpu.CompilerParams(
            use_tc_tiling_on_sc=False, needs_layout_passes=False
        ),
    )
    def emb_gather(idx_hbm, table_hbm, out_hbm, idx_v, rows, trs, sem_g, sem_w):
        wid = lax.axis_index("s") * NC + lax.axis_index("c")
        u0 = wid * U_PER_W
        pltpu.sync_copy(idx_hbm.at[wid], idx_v)

        trow = [kk * 16 + lax.iota(jnp.int32, 16) for kk in range(8)]

        def fire_gather(j, b):
            pltpu.async_copy(table_hbm.at[idx_v.at[j]], rows[b], sem_g[b])

        def wait_gather(b):
            pltpu.make_async_copy(
                table_hbm.at[idx_v.at[0]], rows[b], sem_g[b]
            ).wait()

        def fire_writeback(u, b):
            h = u // NBLK
            c = u % NBLK
            pltpu.async_copy(trs[b], out_hbm.at[h, :, c], sem_w[b])

        def wait_writeback(b):
            pltpu.make_async_copy(trs[b], out_hbm.at[0, :, 0], sem_w[b]).wait()

        def transpose_unit(b):
            @plsc.parallel_loop(0, 64, unroll=4)
            def d_body(d):
                tcol = lax.broadcast(d, (16,))
                r = d // 8
                s = d % 8
                for kk in range(8):
                    vals = plsc.load_gather(rows[b], [trow[kk], tcol])
                    trs[b][r, pl.ds(s * 128 + kk * 16, 16)] = vals

        fire_gather(0, 0)
        fire_gather(1, 1)

        def pair_body(g, carry):
            for b in range(2):
                j = g * 2 + b
                wait_gather(b)

                @pl.when(j >= 2)
                def _():
                    wait_writeback(b)

                transpose_unit(b)
                fire_writeback(u0 + j, b)

                @pl.when(j + 2 < U_PER_W)
                def _():
                    fire_gather(j + 2, b)
            return carry

        lax.fori_loop(0, U_PER_W // 2, pair_body, 0)
        wait_writeback(0)
        wait_writeback(1)

    return emb_gather


_emb_gather = _make_kernel()


@jax.jit
def kernel(token_ids, lookup_table):
    tbl_pad = jnp.pad(lookup_table, ((0, 0), (0, 64)))
    tok3 = token_ids.T.reshape(NW, U_PER_W, CHUNK)
    out5 = _emb_gather(tok3, tbl_pad)
    out = (
        out5.reshape(HIST, 8, NBLK, 8, 128)
        .transpose(2, 4, 0, 1, 3)
        .reshape(BATCH, HIST, EMBED_DIM)
    )
    return out
